# Initial kernel scaffold; baseline (speedup 1.0000x reference)
import jax
import jax.numpy as jnp
from jax.experimental import pallas as pl


def kernel(hidden_states, router_logits, w13_weight, w2_weight):
    raise NotImplementedError("write your pallas kernel here")


# speedup vs baseline: 1.6336x; 1.6336x over previous
import functools

import jax
import jax.numpy as jnp
from jax import lax
from jax.experimental import pallas as pl
from jax.experimental.pallas import tpu as pltpu

_TOP_K = 2
_TILE_M = 512
_TILE_N = 512


def _gmm_body(gids, tids, rs, re, fst, x_ref, wg_ref, wu_ref, w2_ref,
              out_ref, acc_ref, *, n_steps, tile_m):
    i = pl.program_id(0)
    j = pl.program_id(1)

    @pl.when(j == 0)
    def _():
        acc_ref[...] = jnp.zeros_like(acc_ref)

    x = x_ref[...]
    wg = wg_ref[0]
    wu = wu_ref[0]
    dn = (((1,), (1,)), ((), ()))
    gate = lax.dot_general(x, wg, dn, preferred_element_type=jnp.float32)
    up = lax.dot_general(x, wu, dn, preferred_element_type=jnp.float32)
    act = gate * jax.nn.sigmoid(gate) * up
    w2b = w2_ref[0]
    acc_ref[...] += lax.dot_general(act, w2b, dn,
                                    preferred_element_type=jnp.float32)

    @pl.when(j == n_steps - 1)
    def _():
        row = (lax.broadcasted_iota(jnp.int32, (tile_m, 1), 0)
               + tids[i] * tile_m)
        mask = (row >= rs[i]) & (row < re[i])
        prev = jnp.where(fst[i] == 1, jnp.zeros_like(acc_ref[...]),
                         out_ref[...])
        out_ref[...] = jnp.where(mask, acc_ref[...], prev)


def _grouped_mlp(xs, w13, w2, gids, tids, rs, re, fst, *, items):
    rows, hidden = xs.shape
    n_exp, two_inter, _ = w13.shape
    inter = two_inter // 2
    n_steps = inter // _TILE_N
    jblk = inter // _TILE_N

    grid_spec = pltpu.PrefetchScalarGridSpec(
        num_scalar_prefetch=5,
        grid=(items, n_steps),
        in_specs=[
            pl.BlockSpec((_TILE_M, hidden),
                         lambda i, j, g, t, s, e, f: (t[i], 0)),
            pl.BlockSpec((1, _TILE_N, hidden),
                         lambda i, j, g, t, s, e, f: (g[i], j, 0)),
            pl.BlockSpec((1, _TILE_N, hidden),
                         lambda i, j, g, t, s, e, f, _jb=jblk: (g[i], _jb + j, 0)),
            pl.BlockSpec((1, hidden, _TILE_N),
                         lambda i, j, g, t, s, e, f: (g[i], 0, j)),
        ],
        out_specs=pl.BlockSpec((_TILE_M, hidden),
                               lambda i, j, g, t, s, e, f: (t[i], 0)),
        scratch_shapes=[pltpu.VMEM((_TILE_M, hidden), jnp.float32)],
    )
    body = functools.partial(_gmm_body, n_steps=n_steps, tile_m=_TILE_M)
    return pl.pallas_call(
        body,
        grid_spec=grid_spec,
        out_shape=jax.ShapeDtypeStruct((rows, hidden), jnp.float32),
    )(gids, tids, rs, re, fst, xs, w13, w13, w2)


def kernel(hidden_states, router_logits, w13_weight, w2_weight):
    t_tokens, hidden = hidden_states.shape
    n_exp = router_logits.shape[-1]
    rows = t_tokens * _TOP_K
    m_tiles = rows // _TILE_M
    items = m_tiles + n_exp - 1

    probs = jax.nn.softmax(router_logits.astype(jnp.float32), axis=-1)
    topk_w, topk_ids = lax.top_k(probs, _TOP_K)
    topk_w = topk_w / jnp.sum(topk_w, axis=-1, keepdims=True)

    eids = topk_ids.reshape(-1).astype(jnp.int32)
    order = jnp.argsort(eids, stable=True)
    token_of_slot = (order // _TOP_K).astype(jnp.int32)
    counts = jnp.sum(eids[None, :] == jnp.arange(n_exp, dtype=jnp.int32)[:, None],
                     axis=1).astype(jnp.int32)
    starts = jnp.concatenate([jnp.zeros((1,), jnp.int32),
                              jnp.cumsum(counts)]).astype(jnp.int32)
    tile_edges = jnp.arange(m_tiles, dtype=jnp.int32) * _TILE_M
    edges = jnp.sort(jnp.concatenate([tile_edges, starts[1:n_exp]]))
    edges_hi = jnp.concatenate([edges[1:],
                                jnp.full((1,), rows, jnp.int32)])
    tids = jnp.minimum(edges // _TILE_M, m_tiles - 1).astype(jnp.int32)
    gids = jnp.minimum(
        jnp.sum(starts[1:n_exp][None, :] <= edges[:, None], axis=1),
        n_exp - 1).astype(jnp.int32)
    fst = jnp.concatenate([jnp.ones((1,), jnp.int32),
                           (tids[1:] != tids[:-1]).astype(jnp.int32)])

    xs = jnp.take(hidden_states, token_of_slot, axis=0)
    ys = _grouped_mlp(xs, w13_weight, w2_weight,
                      gids, tids, edges, edges_hi, fst, items=items)

    pos = jnp.zeros((rows,), jnp.int32).at[order].set(
        jnp.arange(rows, dtype=jnp.int32))
    ys_tk = jnp.take(ys, pos.reshape(t_tokens, _TOP_K).reshape(-1), axis=0)
    ys_tk = ys_tk.reshape(t_tokens, _TOP_K, hidden)
    return jnp.einsum('tk,tkh->th', topk_w, ys_tk)


# speedup vs baseline: 2.0525x; 1.2564x over previous
import functools

import jax
import jax.numpy as jnp
from jax import lax
from jax.experimental import pallas as pl
from jax.experimental.pallas import tpu as pltpu
from jax.experimental.pallas import tpu_sc as plsc

_NC = 2
_NS = 16
_NW = _NC * _NS
_TOP_K = 2
_TILE_M = 512
_TILE_N = 512


def _sc_gather(table, idx, rows, chunk):
    _, hidden = table.shape
    per_w = rows // _NW
    n_chunks = per_w // chunk
    mesh = plsc.VectorSubcoreMesh(core_axis_name="c", subcore_axis_name="s")

    @functools.partial(
        pl.kernel, mesh=mesh,
        out_type=jax.ShapeDtypeStruct((rows, hidden), jnp.float32),
        scratch_types=[
            pltpu.VMEM((chunk,), jnp.int32),
            pltpu.VMEM((chunk, hidden), jnp.float32),
            pltpu.SemaphoreType.DMA,
        ],
    )
    def k(table_hbm, idx_hbm, out_hbm, idx_v, rows_v, sem):
        wid = lax.axis_index("s") * _NC + lax.axis_index("c")
        base = wid * per_w
        for c in range(n_chunks):
            off = base + c * chunk
            pltpu.sync_copy(idx_hbm.at[pl.ds(off, chunk)], idx_v)
            pltpu.async_copy(table_hbm.at[idx_v], rows_v, sem).wait()
            pltpu.sync_copy(rows_v, out_hbm.at[pl.ds(off, chunk)])

    return k(table, idx)


def _sc_combine(ys, pos_a, pos_b, tokens, chunk):
    _, hidden = ys.shape
    nvec = hidden // 16
    per_w = tokens // _NW
    n_chunks = per_w // chunk
    mesh = plsc.VectorSubcoreMesh(core_axis_name="c", subcore_axis_name="s")

    @functools.partial(
        pl.kernel, mesh=mesh,
        out_type=jax.ShapeDtypeStruct((tokens, hidden), jnp.float32),
        scratch_types=[
            pltpu.VMEM((chunk,), jnp.int32),
            pltpu.VMEM((chunk,), jnp.int32),
            pltpu.VMEM((chunk, hidden), jnp.float32),
            pltpu.VMEM((chunk, hidden), jnp.float32),
            pltpu.VMEM((chunk, hidden), jnp.float32),
            pltpu.SemaphoreType.DMA,
            pltpu.SemaphoreType.DMA,
        ],
    )
    def k(ys_hbm, pa_hbm, pb_hbm, out_hbm, ia_v, ib_v, ra_v, rb_v, out_v,
          sem_a, sem_b):
        wid = lax.axis_index("s") * _NC + lax.axis_index("c")
        base = wid * per_w
        for c in range(n_chunks):
            off = base + c * chunk
            pltpu.sync_copy(pa_hbm.at[pl.ds(off, chunk)], ia_v)
            pltpu.sync_copy(pb_hbm.at[pl.ds(off, chunk)], ib_v)
            cp_a = pltpu.async_copy(ys_hbm.at[ia_v], ra_v, sem_a)
            cp_b = pltpu.async_copy(ys_hbm.at[ib_v], rb_v, sem_b)
            cp_a.wait()
            cp_b.wait()

            def row(r, carry):
                for j in range(nvec):
                    sl = pl.ds(j * 16, 16)
                    out_v[r, sl] = ra_v[r, sl] + rb_v[r, sl]
                return carry

            lax.fori_loop(0, chunk, row, 0)
            pltpu.sync_copy(out_v, out_hbm.at[pl.ds(off, chunk)])

    return k(ys, pos_a, pos_b)


def _gmm_body(gids, tids, rs, re, fst, x_ref, wg_ref, wu_ref, w2_ref, ws_ref,
              out_ref, acc_ref, *, n_steps, tile_m):
    i = pl.program_id(0)
    j = pl.program_id(1)

    @pl.when(j == 0)
    def _():
        acc_ref[...] = jnp.zeros_like(acc_ref)

    x = x_ref[...]
    wg = wg_ref[0]
    wu = wu_ref[0]
    dn = (((1,), (1,)), ((), ()))
    gate = lax.dot_general(x, wg, dn, preferred_element_type=jnp.float32)
    up = lax.dot_general(x, wu, dn, preferred_element_type=jnp.float32)
    act = gate * jax.nn.sigmoid(gate) * up
    w2b = w2_ref[0]
    acc_ref[...] += lax.dot_general(act, w2b, dn,
                                    preferred_element_type=jnp.float32)

    @pl.when(j == n_steps - 1)
    def _():
        row = (lax.broadcasted_iota(jnp.int32, (tile_m, 1), 0)
               + tids[i] * tile_m)
        mask = (row >= rs[i]) & (row < re[i])
        wv = ws_ref[:, 0:1]
        prev = jnp.where(fst[i] == 1, jnp.zeros_like(acc_ref[...]),
                         out_ref[...])
        out_ref[...] = jnp.where(mask, acc_ref[...] * wv, prev)


def _grouped_mlp(xs, w13, w2, wb, gids, tids, rs, re, fst, *, items):
    rows, hidden = xs.shape
    n_exp, two_inter, _ = w13.shape
    inter = two_inter // 2
    n_steps = inter // _TILE_N
    jblk = inter // _TILE_N

    grid_spec = pltpu.PrefetchScalarGridSpec(
        num_scalar_prefetch=5,
        grid=(items, n_steps),
        in_specs=[
            pl.BlockSpec((_TILE_M, hidden),
                         lambda i, j, g, t, s, e, f: (t[i], 0)),
            pl.BlockSpec((1, _TILE_N, hidden),
                         lambda i, j, g, t, s, e, f: (g[i], j, 0)),
            pl.BlockSpec((1, _TILE_N, hidden),
                         lambda i, j, g, t, s, e, f, _jb=jblk: (g[i], _jb + j, 0)),
            pl.BlockSpec((1, hidden, _TILE_N),
                         lambda i, j, g, t, s, e, f: (g[i], 0, j)),
            pl.BlockSpec((_TILE_M, 128),
                         lambda i, j, g, t, s, e, f: (t[i], 0)),
        ],
        out_specs=pl.BlockSpec((_TILE_M, hidden),
                               lambda i, j, g, t, s, e, f: (t[i], 0)),
        scratch_shapes=[pltpu.VMEM((_TILE_M, hidden), jnp.float32)],
    )
    body = functools.partial(_gmm_body, n_steps=n_steps, tile_m=_TILE_M)
    return pl.pallas_call(
        body,
        grid_spec=grid_spec,
        out_shape=jax.ShapeDtypeStruct((rows, hidden), jnp.float32),
    )(gids, tids, rs, re, fst, xs, w13, w13, w2, wb)


def kernel(hidden_states, router_logits, w13_weight, w2_weight):
    t_tokens, hidden = hidden_states.shape
    n_exp = router_logits.shape[-1]
    rows = t_tokens * _TOP_K
    m_tiles = rows // _TILE_M
    items = m_tiles + n_exp - 1

    probs = jax.nn.softmax(router_logits.astype(jnp.float32), axis=-1)
    topk_w, topk_ids = lax.top_k(probs, _TOP_K)
    topk_w = topk_w / jnp.sum(topk_w, axis=-1, keepdims=True)

    eids = topk_ids.reshape(-1).astype(jnp.int32)
    order = jnp.argsort(eids, stable=True)
    token_of_slot = (order // _TOP_K).astype(jnp.int32)
    counts = jnp.sum(eids[None, :] == jnp.arange(n_exp, dtype=jnp.int32)[:, None],
                     axis=1).astype(jnp.int32)
    starts = jnp.concatenate([jnp.zeros((1,), jnp.int32),
                              jnp.cumsum(counts)]).astype(jnp.int32)
    tile_edges = jnp.arange(m_tiles, dtype=jnp.int32) * _TILE_M
    edges = jnp.sort(jnp.concatenate([tile_edges, starts[1:n_exp]]))
    edges_hi = jnp.concatenate([edges[1:],
                                jnp.full((1,), rows, jnp.int32)])
    tids = jnp.minimum(edges // _TILE_M, m_tiles - 1).astype(jnp.int32)
    gids = jnp.minimum(
        jnp.sum(starts[1:n_exp][None, :] <= edges[:, None], axis=1),
        n_exp - 1).astype(jnp.int32)
    fst = jnp.concatenate([jnp.ones((1,), jnp.int32),
                           (tids[1:] != tids[:-1]).astype(jnp.int32)])

    xs = _sc_gather(hidden_states, token_of_slot, rows, chunk=64)

    w_slot = topk_w.reshape(-1)[order]
    wb = jnp.broadcast_to(w_slot[:, None], (rows, 128))
    ys = _grouped_mlp(xs, w13_weight, w2_weight, wb,
                      gids, tids, edges, edges_hi, fst, items=items)

    pos = jnp.zeros((rows,), jnp.int32).at[order].set(
        jnp.arange(rows, dtype=jnp.int32))
    pos_tk = pos.reshape(t_tokens, _TOP_K)
    return _sc_combine(ys, pos_tk[:, 0], pos_tk[:, 1], t_tokens, chunk=32)


# speedup vs baseline: 2.2766x; 1.1092x over previous
import functools

import jax
import jax.numpy as jnp
from jax import lax
from jax.experimental import pallas as pl
from jax.experimental.pallas import tpu as pltpu
from jax.experimental.pallas import tpu_sc as plsc

_NC = 2
_NS = 16
_NW = _NC * _NS
_TOP_K = 2
_TILE_M = 512
_TILE_N = 1024


def _sc_gather(table, idx, rows, chunk):
    _, hidden = table.shape
    per_w = rows // _NW
    n_chunks = per_w // chunk
    mesh = plsc.VectorSubcoreMesh(core_axis_name="c", subcore_axis_name="s")

    @functools.partial(
        pl.kernel, mesh=mesh,
        out_type=jax.ShapeDtypeStruct((rows, hidden), jnp.float32),
        scratch_types=[
            pltpu.VMEM((chunk,), jnp.int32),
            pltpu.VMEM((chunk, hidden), jnp.float32),
            pltpu.SemaphoreType.DMA,
        ],
    )
    def k(table_hbm, idx_hbm, out_hbm, idx_v, rows_v, sem):
        wid = lax.axis_index("s") * _NC + lax.axis_index("c")
        base = wid * per_w
        for c in range(n_chunks):
            off = base + c * chunk
            pltpu.sync_copy(idx_hbm.at[pl.ds(off, chunk)], idx_v)
            pltpu.async_copy(table_hbm.at[idx_v], rows_v, sem).wait()
            pltpu.sync_copy(rows_v, out_hbm.at[pl.ds(off, chunk)])

    return k(table, idx)


def _sc_combine(ys, pos_a, pos_b, tokens, chunk):
    _, hidden = ys.shape
    nvec = hidden // 16
    per_w = tokens // _NW
    n_chunks = per_w // chunk
    mesh = plsc.VectorSubcoreMesh(core_axis_name="c", subcore_axis_name="s")

    @functools.partial(
        pl.kernel, mesh=mesh,
        out_type=jax.ShapeDtypeStruct((tokens, hidden), jnp.float32),
        scratch_types=[
            pltpu.VMEM((chunk,), jnp.int32),
            pltpu.VMEM((chunk,), jnp.int32),
            pltpu.VMEM((chunk, hidden), jnp.float32),
            pltpu.VMEM((chunk, hidden), jnp.float32),
            pltpu.VMEM((chunk, hidden), jnp.float32),
            pltpu.SemaphoreType.DMA,
            pltpu.SemaphoreType.DMA,
        ],
    )
    def k(ys_hbm, pa_hbm, pb_hbm, out_hbm, ia_v, ib_v, ra_v, rb_v, out_v,
          sem_a, sem_b):
        wid = lax.axis_index("s") * _NC + lax.axis_index("c")
        base = wid * per_w
        for c in range(n_chunks):
            off = base + c * chunk
            pltpu.sync_copy(pa_hbm.at[pl.ds(off, chunk)], ia_v)
            pltpu.sync_copy(pb_hbm.at[pl.ds(off, chunk)], ib_v)
            cp_a = pltpu.async_copy(ys_hbm.at[ia_v], ra_v, sem_a)
            cp_b = pltpu.async_copy(ys_hbm.at[ib_v], rb_v, sem_b)
            cp_a.wait()
            cp_b.wait()

            def row(r, carry):
                for j in range(nvec):
                    sl = pl.ds(j * 16, 16)
                    out_v[r, sl] = ra_v[r, sl] + rb_v[r, sl]
                return carry

            lax.fori_loop(0, chunk, row, 0)
            pltpu.sync_copy(out_v, out_hbm.at[pl.ds(off, chunk)])

    return k(ys, pos_a, pos_b)


def _gmm_body(gids, tids, rs, re, fst, x_ref, wg_ref, wu_ref, w2_ref, ws_ref,
              out_ref, acc_ref, *, n_steps, tile_m):
    i = pl.program_id(0)
    j = pl.program_id(1)

    @pl.when(j == 0)
    def _():
        acc_ref[...] = jnp.zeros_like(acc_ref)

    x = x_ref[...]
    wg = wg_ref[0]
    wu = wu_ref[0]
    dn = (((1,), (1,)), ((), ()))
    gate = lax.dot_general(x, wg, dn, preferred_element_type=jnp.float32)
    up = lax.dot_general(x, wu, dn, preferred_element_type=jnp.float32)
    act = gate * jax.nn.sigmoid(gate) * up
    w2b = w2_ref[0]
    acc_ref[...] += lax.dot_general(act, w2b, dn,
                                    preferred_element_type=jnp.float32)

    @pl.when(j == n_steps - 1)
    def _():
        row = (lax.broadcasted_iota(jnp.int32, (tile_m, 1), 0)
               + tids[i] * tile_m)
        mask = (row >= rs[i]) & (row < re[i])
        wv = ws_ref[:, 0:1]
        prev = jnp.where(fst[i] == 1, jnp.zeros_like(acc_ref[...]),
                         out_ref[...])
        out_ref[...] = jnp.where(mask, acc_ref[...] * wv, prev)


def _grouped_mlp(xs, w13, w2, wb, gids, tids, rs, re, fst, *, items):
    rows, hidden = xs.shape
    n_exp, two_inter, _ = w13.shape
    inter = two_inter // 2
    n_steps = inter // _TILE_N
    jblk = inter // _TILE_N

    grid_spec = pltpu.PrefetchScalarGridSpec(
        num_scalar_prefetch=5,
        grid=(items, n_steps),
        in_specs=[
            pl.BlockSpec((_TILE_M, hidden),
                         lambda i, j, g, t, s, e, f: (t[i], 0)),
            pl.BlockSpec((1, _TILE_N, hidden),
                         lambda i, j, g, t, s, e, f: (g[i], j, 0)),
            pl.BlockSpec((1, _TILE_N, hidden),
                         lambda i, j, g, t, s, e, f, _jb=jblk: (g[i], _jb + j, 0)),
            pl.BlockSpec((1, hidden, _TILE_N),
                         lambda i, j, g, t, s, e, f: (g[i], 0, j)),
            pl.BlockSpec((_TILE_M, 128),
                         lambda i, j, g, t, s, e, f: (t[i], 0)),
        ],
        out_specs=pl.BlockSpec((_TILE_M, hidden),
                               lambda i, j, g, t, s, e, f: (t[i], 0)),
        scratch_shapes=[pltpu.VMEM((_TILE_M, hidden), jnp.float32)],
    )
    body = functools.partial(_gmm_body, n_steps=n_steps, tile_m=_TILE_M)
    return pl.pallas_call(
        body,
        grid_spec=grid_spec,
        out_shape=jax.ShapeDtypeStruct((rows, hidden), jnp.float32),
    )(gids, tids, rs, re, fst, xs, w13, w13, w2, wb)


def kernel(hidden_states, router_logits, w13_weight, w2_weight):
    t_tokens, hidden = hidden_states.shape
    n_exp = router_logits.shape[-1]
    rows = t_tokens * _TOP_K
    m_tiles = rows // _TILE_M
    items = m_tiles + n_exp - 1

    probs = jax.nn.softmax(router_logits.astype(jnp.float32), axis=-1)
    topk_w, topk_ids = lax.top_k(probs, _TOP_K)
    topk_w = topk_w / jnp.sum(topk_w, axis=-1, keepdims=True)

    eids = topk_ids.reshape(-1).astype(jnp.int32)
    order = jnp.argsort(eids, stable=True)
    token_of_slot = (order // _TOP_K).astype(jnp.int32)
    counts = jnp.sum(eids[None, :] == jnp.arange(n_exp, dtype=jnp.int32)[:, None],
                     axis=1).astype(jnp.int32)
    starts = jnp.concatenate([jnp.zeros((1,), jnp.int32),
                              jnp.cumsum(counts)]).astype(jnp.int32)
    tile_edges = jnp.arange(m_tiles, dtype=jnp.int32) * _TILE_M
    edges = jnp.sort(jnp.concatenate([tile_edges, starts[1:n_exp]]))
    edges_hi = jnp.concatenate([edges[1:],
                                jnp.full((1,), rows, jnp.int32)])
    tids = jnp.minimum(edges // _TILE_M, m_tiles - 1).astype(jnp.int32)
    gids = jnp.minimum(
        jnp.sum(starts[1:n_exp][None, :] <= edges[:, None], axis=1),
        n_exp - 1).astype(jnp.int32)
    fst = jnp.concatenate([jnp.ones((1,), jnp.int32),
                           (tids[1:] != tids[:-1]).astype(jnp.int32)])

    xs = _sc_gather(hidden_states, token_of_slot, rows, chunk=64)

    w_slot = topk_w.reshape(-1)[order]
    wb = jnp.broadcast_to(w_slot[:, None], (rows, 128))
    ys = _grouped_mlp(xs, w13_weight, w2_weight, wb,
                      gids, tids, edges, edges_hi, fst, items=items)

    pos = jnp.zeros((rows,), jnp.int32).at[order].set(
        jnp.arange(rows, dtype=jnp.int32))
    pos_tk = pos.reshape(t_tokens, _TOP_K)
    return _sc_combine(ys, pos_tk[:, 0], pos_tk[:, 1], t_tokens, chunk=32)
